# Initial kernel scaffold; baseline (speedup 1.0000x reference)
import jax
import jax.numpy as jnp
from jax.experimental import pallas as pl


def kernel(x, edge_index, edge_attr, W_node, W_attn, W_to_node, W_edge):
    raise NotImplementedError("write your pallas kernel here")


# speedup vs baseline: 1.2293x; 1.2293x over previous
import functools

import jax
import jax.numpy as jnp
from jax.experimental import pallas as pl
from jax.experimental.pallas import tpu as pltpu

N = 10000
E = 320000
D = 128
EA = 16

_NBLK = 2000
_EBLK = 8000


def _node_mm_body(x_ref, wn_ref, w1_ref, wa_ref, z_ref, t_ref, a_ref):
    z = jnp.dot(x_ref[...], wn_ref[...], preferred_element_type=jnp.float32)
    z_ref[...] = z
    t_ref[...] = jnp.dot(z, w1_ref[...], preferred_element_type=jnp.float32)
    a_ref[...] = jnp.dot(z, wa_ref[...], preferred_element_type=jnp.float32)


def _node_mms(x, W_node, W1, Wa):
    return pl.pallas_call(
        _node_mm_body,
        grid=(N // _NBLK,),
        in_specs=[
            pl.BlockSpec((_NBLK, D), lambda i: (i, 0)),
            pl.BlockSpec((D, D), lambda i: (0, 0)),
            pl.BlockSpec((D, D), lambda i: (0, 0)),
            pl.BlockSpec((D, 2), lambda i: (0, 0)),
        ],
        out_specs=[
            pl.BlockSpec((_NBLK, D), lambda i: (i, 0)),
            pl.BlockSpec((_NBLK, D), lambda i: (i, 0)),
            pl.BlockSpec((_NBLK, 2), lambda i: (i, 0)),
        ],
        out_shape=[
            jax.ShapeDtypeStruct((N, D), jnp.float32),
            jax.ShapeDtypeStruct((N, D), jnp.float32),
            jax.ShapeDtypeStruct((N, 2), jnp.float32),
        ],
    )(x, W_node, W1, Wa)


def _edge_mm_body(ea_ref, wa3_ref, we3_ref, ee_ref, wet_ref):
    ea = ea_ref[...]
    ee_ref[...] = jnp.dot(ea, wa3_ref[...], preferred_element_type=jnp.float32)
    wet_ref[...] = jnp.dot(ea, we3_ref[...], preferred_element_type=jnp.float32)


def _edge_mms(edge_attr, wa3, We3):
    return pl.pallas_call(
        _edge_mm_body,
        grid=(E // _EBLK,),
        in_specs=[
            pl.BlockSpec((_EBLK, EA), lambda i: (i, 0)),
            pl.BlockSpec((EA, 1), lambda i: (0, 0)),
            pl.BlockSpec((EA, EA), lambda i: (0, 0)),
        ],
        out_specs=[
            pl.BlockSpec((_EBLK, 1), lambda i: (i, 0)),
            pl.BlockSpec((_EBLK, EA), lambda i: (i, 0)),
        ],
        out_shape=[
            jax.ShapeDtypeStruct((E, 1), jnp.float32),
            jax.ShapeDtypeStruct((E, EA), jnp.float32),
        ],
    )(edge_attr, wa3, We3)


def _combine_body(hw_ref, s16_ref, den_ref, deg_ref, z_ref, w2_ref, we1_ref,
                  we2_ref, h_ref, us_ref, ud_ref):
    hw = hw_ref[0] + hw_ref[1]
    s16 = s16_ref[0] + s16_ref[1]
    den = jnp.sum(den_ref[...], axis=1)
    deg = jnp.sum(deg_ref[...], axis=1)
    h_agg = (hw + jnp.dot(s16, w2_ref[...], preferred_element_type=jnp.float32))
    h_agg = h_agg / jnp.where(den > 0, den, 1.0)[:, None]
    h = jnp.where((deg > 0)[:, None], h_agg, z_ref[...])
    h_ref[...] = h
    us_ref[...] = jnp.dot(h, we1_ref[...], preferred_element_type=jnp.float32)
    ud_ref[...] = jnp.dot(h, we2_ref[...], preferred_element_type=jnp.float32)


def _combine(hw_p, s16_p, den_p, deg_p, z, W2, We1, We2):
    nparts = den_p.shape[1]
    return pl.pallas_call(
        _combine_body,
        grid=(N // _NBLK,),
        in_specs=[
            pl.BlockSpec((2, _NBLK, D), lambda i: (0, i, 0)),
            pl.BlockSpec((2, _NBLK, EA), lambda i: (0, i, 0)),
            pl.BlockSpec((_NBLK, nparts), lambda i: (i, 0)),
            pl.BlockSpec((_NBLK, nparts), lambda i: (i, 0)),
            pl.BlockSpec((_NBLK, D), lambda i: (i, 0)),
            pl.BlockSpec((EA, D), lambda i: (0, 0)),
            pl.BlockSpec((D, EA), lambda i: (0, 0)),
            pl.BlockSpec((D, EA), lambda i: (0, 0)),
        ],
        out_specs=[
            pl.BlockSpec((_NBLK, D), lambda i: (i, 0)),
            pl.BlockSpec((_NBLK, EA), lambda i: (i, 0)),
            pl.BlockSpec((_NBLK, EA), lambda i: (i, 0)),
        ],
        out_shape=[
            jax.ShapeDtypeStruct((N, D), jnp.float32),
            jax.ShapeDtypeStruct((N, EA), jnp.float32),
            jax.ShapeDtypeStruct((N, EA), jnp.float32),
        ],
    )(hw_p, s16_p, den_p, deg_p, z, W2, We1, We2)


def kernel(x, edge_index, edge_attr, W_node, W_attn, W_to_node, W_edge):
    src = edge_index[0].astype(jnp.int32)
    dst = edge_index[1].astype(jnp.int32)
    Wa = W_attn[: 2 * D].reshape(2, D, 1)[:, :, 0].T
    z, t_node, a2 = _node_mms(x, W_node, W_to_node[:D], Wa)
    ee, w_et = _edge_mms(edge_attr, W_attn[2 * D :], W_edge[2 * D :])
    a_src = a2[:, 0]
    a_dst = a2[:, 1]

    e = a_src[src] + a_dst[dst] + ee[:, 0]
    e = jnp.where(e >= 0, e, 0.1 * e)
    p = jnp.exp(e)
    denom = jax.ops.segment_sum(p, dst, num_segments=N)
    deg = jax.ops.segment_sum(jnp.ones_like(p), dst, num_segments=N)
    hw = jax.ops.segment_sum(p[:, None] * t_node[src], dst, num_segments=N)
    s16 = jax.ops.segment_sum(p[:, None] * edge_attr, dst, num_segments=N)
    hw_p = jnp.stack([hw, jnp.zeros_like(hw)])
    s16_p = jnp.stack([s16, jnp.zeros_like(s16)])
    den_p = denom[:, None]
    deg_p = deg[:, None]

    h_new, u_src, u_dst = _combine(hw_p, s16_p, den_p, deg_p, z,
                                   W_to_node[D:], W_edge[:D], W_edge[D : 2 * D])

    w_new = u_src[src] + u_dst[dst] + w_et
    return h_new, w_new


# speedup vs baseline: 5.2942x; 4.3066x over previous
import functools

import jax
import jax.numpy as jnp
from jax import lax
from jax.experimental import pallas as pl
from jax.experimental.pallas import tpu as pltpu
from jax.experimental.pallas import tpu_sc as plsc

N = 10000
E = 320000
D = 128
EA = 16

_NBLK = 2000
_EBLK = 8000

_NC = 2
_NS = 16
_NW = _NC * _NS
_EPW = E // _NW
_C = 80
_NCHUNK = _EPW // _C
_NSH = 10240
_NPS = _NSH // _NS

_SP = 32


def _node_mm_body(x_ref, wn_ref, w1_ref, wa_ref, z_ref, t_ref, a_ref):
    z = jnp.dot(x_ref[...], wn_ref[...], preferred_element_type=jnp.float32)
    z_ref[...] = z
    t = jnp.dot(z, w1_ref[...], preferred_element_type=jnp.float32)
    t_ref[0] = t[:, :D // 2]
    t_ref[1] = t[:, D // 2:]
    a_ref[...] = jnp.dot(z, wa_ref[...], preferred_element_type=jnp.float32)


def _node_mms(x, W_node, W1, Wa):
    return pl.pallas_call(
        _node_mm_body,
        grid=(N // _NBLK,),
        in_specs=[
            pl.BlockSpec((_NBLK, D), lambda i: (i, 0)),
            pl.BlockSpec((D, D), lambda i: (0, 0)),
            pl.BlockSpec((D, D), lambda i: (0, 0)),
            pl.BlockSpec((D, 2), lambda i: (0, 0)),
        ],
        out_specs=[
            pl.BlockSpec((_NBLK, D), lambda i: (i, 0)),
            pl.BlockSpec((_NC, _NBLK, D // 2), lambda i: (0, i, 0)),
            pl.BlockSpec((_NBLK, 2), lambda i: (i, 0)),
        ],
        out_shape=[
            jax.ShapeDtypeStruct((N, D), jnp.float32),
            jax.ShapeDtypeStruct((_NC, N, D // 2), jnp.float32),
            jax.ShapeDtypeStruct((N, 2), jnp.float32),
        ],
    )(x, W_node, W1, Wa)


def _edge_mm_body(ea_ref, wa3_ref, we3_ref, ee_ref, wet_ref):
    ea = ea_ref[...]
    ee_ref[...] = jnp.dot(ea, wa3_ref[...], preferred_element_type=jnp.float32)
    wet_ref[...] = jnp.dot(ea, we3_ref[...], preferred_element_type=jnp.float32)


def _edge_mms(edge_attr, wa3, We3):
    return pl.pallas_call(
        _edge_mm_body,
        grid=(E // _EBLK,),
        in_specs=[
            pl.BlockSpec((_EBLK, EA), lambda i: (i, 0)),
            pl.BlockSpec((EA, 1), lambda i: (0, 0)),
            pl.BlockSpec((EA, EA), lambda i: (0, 0)),
        ],
        out_specs=[
            pl.BlockSpec((_EBLK, 1), lambda i: (i, 0)),
            pl.BlockSpec((_EBLK, EA), lambda i: (i, 0)),
        ],
        out_shape=[
            jax.ShapeDtypeStruct((E, 1), jnp.float32),
            jax.ShapeDtypeStruct((E, EA), jnp.float32),
        ],
    )(edge_attr, wa3, We3)


def _combine_body(hw_ref, sp_ref, z_ref, w2_ref, we1_ref, we2_ref,
                  h_ref, us_ref, ud_ref):
    hw = jnp.concatenate([hw_ref[0], hw_ref[1]], axis=-1)
    sp = sp_ref[0] + sp_ref[1]
    s16 = sp[:, 0:EA]
    den = sp[:, EA]
    h_agg = hw + jnp.dot(s16, w2_ref[...], preferred_element_type=jnp.float32)
    h_agg = h_agg / jnp.where(den > 0, den, 1.0)[:, None]
    h = jnp.where((den > 0)[:, None], h_agg, z_ref[...])
    h_ref[...] = h
    us_ref[...] = jnp.dot(h, we1_ref[...], preferred_element_type=jnp.float32)
    ud_ref[...] = jnp.dot(h, we2_ref[...], preferred_element_type=jnp.float32)


def _combine(hw_p, sp_p, z, W2, We1, We2):
    return pl.pallas_call(
        _combine_body,
        grid=(N // _NBLK,),
        in_specs=[
            pl.BlockSpec((_NC, _NBLK, D // 2), lambda i: (0, i, 0)),
            pl.BlockSpec((_NC, _NBLK, _SP), lambda i: (0, i, 0)),
            pl.BlockSpec((_NBLK, D), lambda i: (i, 0)),
            pl.BlockSpec((EA, D), lambda i: (0, 0)),
            pl.BlockSpec((D, EA), lambda i: (0, 0)),
            pl.BlockSpec((D, EA), lambda i: (0, 0)),
        ],
        out_specs=[
            pl.BlockSpec((_NBLK, D), lambda i: (i, 0)),
            pl.BlockSpec((_NBLK, EA), lambda i: (i, 0)),
            pl.BlockSpec((_NBLK, EA), lambda i: (i, 0)),
        ],
        out_shape=[
            jax.ShapeDtypeStruct((N, D), jnp.float32),
            jax.ShapeDtypeStruct((N, EA), jnp.float32),
            jax.ShapeDtypeStruct((N, EA), jnp.float32),
        ],
    )(hw_p, sp_p, z, W2, We1, We2)


_EPT = E // _NS
_NCH = _EPT // _C
_DH = D // 2


def _edge_pass_body(src_h, dst_h, ee_h, asrc_h, adst_h, tn2_h, ea_h,
                    hwp_h, spp_h,
                    asrc_v, adst_v, src_c, dst_c, ee_c, rows, ea_c, sp_c,
                    hw_sh, sp_sh, sem):
    cid = lax.axis_index("c")
    sid = lax.axis_index("s")

    pltpu.sync_copy(asrc_h, asrc_v)
    pltpu.sync_copy(adst_h, adst_v)

    zz = jnp.zeros((16,), jnp.float32)
    for r in range(_C):
        for q in range(_DH // 16):
            rows[r, pl.ds(q * 16, 16)] = zz
        for q in range(_SP // 16):
            sp_c[r, pl.ds(q * 16, 16)] = zz
    for k in range(_NPS // _C):
        pltpu.sync_copy(rows, hw_sh.at[pl.ds(sid * _NPS + k * _C, _C), :])
        pltpu.sync_copy(sp_c, sp_sh.at[pl.ds(sid * _NPS + k * _C, _C), :])
    plsc.subcore_barrier()

    def chunk(k, carry):
        base = sid * _EPT + k * _C
        pltpu.sync_copy(src_h.at[pl.ds(base, _C)], src_c)
        pltpu.sync_copy(dst_h.at[pl.ds(base, _C)], dst_c)
        pltpu.sync_copy(ee_h.at[pl.ds(base, _C)], ee_c)
        pltpu.sync_copy(ea_h.at[pl.ds(base, _C), :], ea_c)
        pltpu.async_copy(tn2_h.at[cid].at[src_c], rows, sem).wait()
        for j in range(_C // 16):
            sv = src_c[pl.ds(j * 16, 16)]
            dv = dst_c[pl.ds(j * 16, 16)]
            av = plsc.load_gather(asrc_v, [sv])
            bv = plsc.load_gather(adst_v, [dv])
            ev = av + bv + ee_c[pl.ds(j * 16, 16)]
            ev = jnp.where(ev >= 0.0, ev, 0.1 * ev)
            pv = jnp.exp(ev)
            for l in range(16):
                e = j * 16 + l
                pvb = jnp.full((16,), pv[l], jnp.float32)
                for q in range(_DH // 16):
                    rows[e, pl.ds(q * 16, 16)] = rows[e, pl.ds(q * 16, 16)] * pvb
                sp_c[e, pl.ds(0, 16)] = ea_c[e, :] * pvb
                sp_c[e, pl.ds(16, 16)] = pvb
        pltpu.sync_copy(rows, hw_sh.at[dst_c], add=True)

        @pl.when(lax.rem(k, 2) == cid)
        def _():
            pltpu.sync_copy(sp_c, sp_sh.at[dst_c], add=True)

        return carry

    lax.fori_loop(0, _NCH, chunk, 0)
    plsc.subcore_barrier()

    pltpu.sync_copy(hw_sh.at[pl.ds(sid * _NPS, _NPS), :],
                    hwp_h.at[cid, pl.ds(sid * _NPS, _NPS), :])
    pltpu.sync_copy(sp_sh.at[pl.ds(sid * _NPS, _NPS), :],
                    spp_h.at[cid, pl.ds(sid * _NPS, _NPS), :])


def _edge_pass(src, dst, ee, a_src, a_dst, t2, edge_attr):
    mesh = plsc.VectorSubcoreMesh(core_axis_name="c", subcore_axis_name="s")
    f = pl.kernel(
        _edge_pass_body,
        out_type=[
            jax.ShapeDtypeStruct((_NC, _NSH, _DH), jnp.float32),
            jax.ShapeDtypeStruct((_NC, _NSH, _SP), jnp.float32),
        ],
        mesh=mesh,
        scratch_types=[
            pltpu.VMEM((N,), jnp.float32),
            pltpu.VMEM((N,), jnp.float32),
            pltpu.VMEM((_C,), jnp.int32),
            pltpu.VMEM((_C,), jnp.int32),
            pltpu.VMEM((_C,), jnp.float32),
            pltpu.VMEM((_C, _DH), jnp.float32),
            pltpu.VMEM((_C, EA), jnp.float32),
            pltpu.VMEM((_C, _SP), jnp.float32),
            pltpu.VMEM_SHARED((_NSH, _DH), jnp.float32),
            pltpu.VMEM_SHARED((_NSH, _SP), jnp.float32),
            pltpu.SemaphoreType.DMA,
        ],
        compiler_params=pltpu.CompilerParams(needs_layout_passes=False,
                                             use_tc_tiling_on_sc=False),
    )
    return f(src, dst, ee, a_src, a_dst, t2, edge_attr)


def _wnew_body(src_h, dst_h, us_h, ud_h, wet_h, wnew_h,
               src_c, dst_c, usr, udr, wet_c, sem, sem2):
    cid = lax.axis_index("c")
    sid = lax.axis_index("s")
    wid = cid * _NS + sid

    def chunk(k, carry):
        base = wid * _EPW + k * _C
        pltpu.sync_copy(src_h.at[pl.ds(base, _C)], src_c)
        pltpu.sync_copy(dst_h.at[pl.ds(base, _C)], dst_c)
        pltpu.sync_copy(wet_h.at[pl.ds(base, _C), :], wet_c)
        ca = pltpu.async_copy(us_h.at[src_c], usr, sem)
        cb = pltpu.async_copy(ud_h.at[dst_c], udr, sem2)
        ca.wait()
        cb.wait()
        for e in range(_C):
            wet_c[e, :] = wet_c[e, :] + usr[e, :] + udr[e, :]
        pltpu.sync_copy(wet_c, wnew_h.at[pl.ds(base, _C), :])
        return carry

    lax.fori_loop(0, _NCHUNK, chunk, 0)


def _wnew(src, dst, u_src, u_dst, w_et):
    mesh = plsc.VectorSubcoreMesh(core_axis_name="c", subcore_axis_name="s")
    f = pl.kernel(
        _wnew_body,
        out_type=jax.ShapeDtypeStruct((E, EA), jnp.float32),
        mesh=mesh,
        scratch_types=[
            pltpu.VMEM((_C,), jnp.int32),
            pltpu.VMEM((_C,), jnp.int32),
            pltpu.VMEM((_C, EA), jnp.float32),
            pltpu.VMEM((_C, EA), jnp.float32),
            pltpu.VMEM((_C, EA), jnp.float32),
            pltpu.SemaphoreType.DMA,
            pltpu.SemaphoreType.DMA,
        ],
        compiler_params=pltpu.CompilerParams(needs_layout_passes=False,
                                             use_tc_tiling_on_sc=False),
    )
    return f(src, dst, u_src, u_dst, w_et)


def kernel(x, edge_index, edge_attr, W_node, W_attn, W_to_node, W_edge):
    src = edge_index[0].astype(jnp.int32)
    dst = edge_index[1].astype(jnp.int32)
    Wa = jnp.concatenate([W_attn[:D], W_attn[D:2 * D]], axis=1)
    z, t2, a2 = _node_mms(x, W_node, W_to_node[:D], Wa)
    ee, w_et = _edge_mms(edge_attr, W_attn[2 * D:], W_edge[2 * D:])
    a_src = a2[:, 0]
    a_dst = a2[:, 1]

    hw_p, sp_p = _edge_pass(src, dst, ee[:, 0], a_src, a_dst, t2, edge_attr)

    h_new, u_src, u_dst = _combine(hw_p, sp_p, z,
                                   W_to_node[D:], W_edge[:D], W_edge[D:2 * D])

    w_new = _wnew(src, dst, u_src, u_dst, w_et)
    return h_new, w_new


# speedup vs baseline: 7.9538x; 1.5024x over previous
import functools

import jax
import jax.numpy as jnp
from jax import lax
from jax.experimental import pallas as pl
from jax.experimental.pallas import tpu as pltpu
from jax.experimental.pallas import tpu_sc as plsc

N = 10000
E = 320000
D = 128
EA = 16

_NBLK = 2000
_EBLK = 8000

_NC = 2
_NS = 16
_NW = _NC * _NS
_EPW = E // _NW
_C = 80
_NCHUNK = _EPW // _C
_NSH = 10240
_NPS = _NSH // _NS

_SP = 32


def _node_mm_body(x_ref, wn_ref, w1_ref, wa_ref, z_ref, t_ref, a_ref):
    z = jnp.dot(x_ref[...], wn_ref[...], preferred_element_type=jnp.float32)
    z_ref[...] = z
    t = jnp.dot(z, w1_ref[...], preferred_element_type=jnp.float32)
    t_ref[0] = t[:, :D // 2]
    t_ref[1] = t[:, D // 2:]
    a_ref[...] = jnp.dot(z, wa_ref[...], preferred_element_type=jnp.float32)


def _node_mms(x, W_node, W1, Wa):
    return pl.pallas_call(
        _node_mm_body,
        grid=(N // _NBLK,),
        in_specs=[
            pl.BlockSpec((_NBLK, D), lambda i: (i, 0)),
            pl.BlockSpec((D, D), lambda i: (0, 0)),
            pl.BlockSpec((D, D), lambda i: (0, 0)),
            pl.BlockSpec((D, 2), lambda i: (0, 0)),
        ],
        out_specs=[
            pl.BlockSpec((_NBLK, D), lambda i: (i, 0)),
            pl.BlockSpec((_NC, _NBLK, D // 2), lambda i: (0, i, 0)),
            pl.BlockSpec((_NBLK, 2), lambda i: (i, 0)),
        ],
        out_shape=[
            jax.ShapeDtypeStruct((N, D), jnp.float32),
            jax.ShapeDtypeStruct((_NC, N, D // 2), jnp.float32),
            jax.ShapeDtypeStruct((N, 2), jnp.float32),
        ],
    )(x, W_node, W1, Wa)


def _edge_mm_body(ea_ref, wa3_ref, we3_ref, ee_ref, wet_ref):
    ea = ea_ref[...]
    ee_ref[...] = jnp.dot(ea, wa3_ref[...], preferred_element_type=jnp.float32)
    wet_ref[...] = jnp.dot(ea, we3_ref[...], preferred_element_type=jnp.float32)


def _edge_mms(edge_attr, wa3, We3):
    return pl.pallas_call(
        _edge_mm_body,
        grid=(E // _EBLK,),
        in_specs=[
            pl.BlockSpec((_EBLK, EA), lambda i: (i, 0)),
            pl.BlockSpec((EA, 1), lambda i: (0, 0)),
            pl.BlockSpec((EA, EA), lambda i: (0, 0)),
        ],
        out_specs=[
            pl.BlockSpec((_EBLK, 1), lambda i: (i, 0)),
            pl.BlockSpec((_EBLK, EA), lambda i: (i, 0)),
        ],
        out_shape=[
            jax.ShapeDtypeStruct((E, 1), jnp.float32),
            jax.ShapeDtypeStruct((E, EA), jnp.float32),
        ],
    )(edge_attr, wa3, We3)


def _combine_body(hw_ref, sp_ref, z_ref, w2_ref, we1_ref, we2_ref,
                  h_ref, us_ref, ud_ref):
    hw = jnp.concatenate([hw_ref[0], hw_ref[1]], axis=-1)
    sp = sp_ref[0] + sp_ref[1]
    s16 = sp[:, 0:EA]
    den = sp[:, EA]
    h_agg = hw + jnp.dot(s16, w2_ref[...], preferred_element_type=jnp.float32)
    h_agg = h_agg / jnp.where(den > 0, den, 1.0)[:, None]
    h = jnp.where((den > 0)[:, None], h_agg, z_ref[...])
    h_ref[...] = h
    us_ref[...] = jnp.dot(h, we1_ref[...], preferred_element_type=jnp.float32)
    ud_ref[...] = jnp.dot(h, we2_ref[...], preferred_element_type=jnp.float32)


def _combine(hw_p, sp_p, z, W2, We1, We2):
    return pl.pallas_call(
        _combine_body,
        grid=(N // _NBLK,),
        in_specs=[
            pl.BlockSpec((_NC, _NBLK, D // 2), lambda i: (0, i, 0)),
            pl.BlockSpec((_NC, _NBLK, _SP), lambda i: (0, i, 0)),
            pl.BlockSpec((_NBLK, D), lambda i: (i, 0)),
            pl.BlockSpec((EA, D), lambda i: (0, 0)),
            pl.BlockSpec((D, EA), lambda i: (0, 0)),
            pl.BlockSpec((D, EA), lambda i: (0, 0)),
        ],
        out_specs=[
            pl.BlockSpec((_NBLK, D), lambda i: (i, 0)),
            pl.BlockSpec((_NBLK, EA), lambda i: (i, 0)),
            pl.BlockSpec((_NBLK, EA), lambda i: (i, 0)),
        ],
        out_shape=[
            jax.ShapeDtypeStruct((N, D), jnp.float32),
            jax.ShapeDtypeStruct((N, EA), jnp.float32),
            jax.ShapeDtypeStruct((N, EA), jnp.float32),
        ],
    )(hw_p, sp_p, z, W2, We1, We2)


_EPT = E // _NS
_C = 80
_NCH = _EPT // _C
_DH = D // 2


def _edge_pass_body(src_h, dst_h, ee_h, asrc_h, adst_h, tn2_h, ea_h,
                    hwp_h, spp_h,
                    asrc_v, adst_v, srcb, dstb, sdst, eeb, eab, gbuf, sbuf,
                    spb, hw_sh, sp_sh, gsem, rsem, psem, lsem, ssem):
    cid = lax.axis_index("c")
    sid = lax.axis_index("s")

    pltpu.sync_copy(asrc_h, asrc_v)
    pltpu.sync_copy(adst_h, adst_v)

    zz = jnp.zeros((16,), jnp.float32)
    for r in range(_C):
        for q in range(_DH // 16):
            sbuf[0, r, pl.ds(q * 16, 16)] = zz
        for q in range(_SP // 16):
            spb[0, r, pl.ds(q * 16, 16)] = zz
    for k in range(_NPS // _C):
        pltpu.sync_copy(sbuf.at[0], hw_sh.at[pl.ds(sid * _NPS + k * _C, _C), :])
        pltpu.sync_copy(spb.at[0], sp_sh.at[pl.ds(sid * _NPS + k * _C, _C), :])
    plsc.subcore_barrier()

    ebase = sid * _EPT

    def start_src(k, b):
        pltpu.make_async_copy(src_h.at[pl.ds(ebase + k * _C, _C)],
                              srcb.at[b], ssem.at[b]).start()

    def drain_src(b):
        pltpu.make_async_copy(src_h.at[pl.ds(0, _C)], srcb.at[b],
                              ssem.at[b]).wait()

    def start_lin(k, b):
        pltpu.make_async_copy(dst_h.at[pl.ds(ebase + k * _C, _C)],
                              dstb.at[b], lsem.at[b]).start()
        pltpu.make_async_copy(ee_h.at[pl.ds(ebase + k * _C, _C)],
                              eeb.at[b], lsem.at[b]).start()
        pltpu.make_async_copy(ea_h.at[pl.ds(ebase + k * _C, _C), :],
                              eab.at[b], lsem.at[b]).start()

    def drain_lin(b):
        pltpu.make_async_copy(dst_h.at[pl.ds(0, _C)], dstb.at[b],
                              lsem.at[b]).wait()
        pltpu.make_async_copy(ee_h.at[pl.ds(0, _C)], eeb.at[b],
                              lsem.at[b]).wait()
        pltpu.make_async_copy(ea_h.at[pl.ds(0, _C), :], eab.at[b],
                              lsem.at[b]).wait()

    def start_gather(bsrc, b):
        pltpu.make_async_copy(tn2_h.at[cid].at[srcb.at[bsrc]],
                              gbuf.at[b], gsem.at[b]).start()

    def drain_gather(b):
        pltpu.make_async_copy(tn2_h.at[cid].at[srcb.at[0]],
                              gbuf.at[b], gsem.at[b]).wait()

    def start_scat(b):
        pltpu.make_async_copy(sbuf.at[b], hw_sh.at[sdst.at[b]],
                              rsem.at[b]).start(add=True)

    def drain_scat(b):
        pltpu.make_async_copy(sbuf.at[b], hw_sh.at[sdst.at[0]],
                              rsem.at[b]).wait()

    def start_spscat(b):
        pltpu.make_async_copy(spb.at[b], sp_sh.at[sdst.at[b]],
                              psem.at[b]).start(add=True)

    def drain_spscat(b):
        pltpu.make_async_copy(spb.at[b], sp_sh.at[sdst.at[0]],
                              psem.at[b]).wait()

    def compute(b):
        for g in range(_C // 16):
            sv = srcb[b, pl.ds(g * 16, 16)]
            dv = dstb[b, pl.ds(g * 16, 16)]
            sdst[b, pl.ds(g * 16, 16)] = dv
            av = plsc.load_gather(asrc_v, [sv])
            bv = plsc.load_gather(adst_v, [dv])
            ev = av + bv + eeb[b, pl.ds(g * 16, 16)]
            ev = jnp.where(ev >= 0.0, ev, 0.1 * ev)
            pv = jnp.exp(ev)
            for l in range(16):
                e = g * 16 + l
                pvb = jnp.full((16,), pv[l], jnp.float32)
                for q in range(_DH // 16):
                    sbuf[b, e, pl.ds(q * 16, 16)] = (
                        gbuf[b, e, pl.ds(q * 16, 16)] * pvb)
                spb[b, e, pl.ds(0, 16)] = eab[b, e, :] * pvb
                spb[b, e, pl.ds(16, 16)] = pvb

    def half(k, b):
        @pl.when(k >= 2)
        def _():
            drain_scat(b)

        @pl.when((k >= 2) & (lax.rem(k, 2) == cid))
        def _():
            drain_spscat(b)

        @pl.when(k + 1 < _NCH)
        def _():
            start_lin(k + 1, 1 - b)

        drain_gather(b)

        @pl.when(k + 1 < _NCH)
        def _():
            drain_src(1 - b)
            start_gather(1 - b, 1 - b)

        drain_lin(b)
        compute(b)
        start_scat(b)

        @pl.when(lax.rem(k, 2) == cid)
        def _():
            start_spscat(b)

        @pl.when(k + 2 < _NCH)
        def _():
            start_src(k + 2, b)

    start_src(0, 0)
    start_src(1, 1)
    start_lin(0, 0)
    drain_src(0)
    start_gather(0, 0)

    def pair(t, carry):
        half(2 * t, 0)
        half(2 * t + 1, 1)
        return carry

    lax.fori_loop(0, _NCH // 2, pair, 0)

    drain_scat((_NCH - 1) % 2)
    drain_scat((_NCH - 2) % 2)

    @pl.when(cid == 0)
    def _():
        drain_spscat(0)

    @pl.when(cid == 1)
    def _():
        drain_spscat(1)

    plsc.subcore_barrier()

    pltpu.sync_copy(hw_sh.at[pl.ds(sid * _NPS, _NPS), :],
                    hwp_h.at[cid, pl.ds(sid * _NPS, _NPS), :])
    pltpu.sync_copy(sp_sh.at[pl.ds(sid * _NPS, _NPS), :],
                    spp_h.at[cid, pl.ds(sid * _NPS, _NPS), :])


def _edge_pass(src, dst, ee, a_src, a_dst, t2, edge_attr):
    mesh = plsc.VectorSubcoreMesh(core_axis_name="c", subcore_axis_name="s")
    f = pl.kernel(
        _edge_pass_body,
        out_type=[
            jax.ShapeDtypeStruct((_NC, _NSH, _DH), jnp.float32),
            jax.ShapeDtypeStruct((_NC, _NSH, _SP), jnp.float32),
        ],
        mesh=mesh,
        scratch_types=[
            pltpu.VMEM((N,), jnp.float32),
            pltpu.VMEM((N,), jnp.float32),
            pltpu.VMEM((2, _C), jnp.int32),
            pltpu.VMEM((2, _C), jnp.int32),
            pltpu.VMEM((2, _C), jnp.int32),
            pltpu.VMEM((2, _C), jnp.float32),
            pltpu.VMEM((2, _C, EA), jnp.float32),
            pltpu.VMEM((2, _C, _DH), jnp.float32),
            pltpu.VMEM((2, _C, _DH), jnp.float32),
            pltpu.VMEM((2, _C, _SP), jnp.float32),
            pltpu.VMEM_SHARED((_NSH, _DH), jnp.float32),
            pltpu.VMEM_SHARED((_NSH, _SP), jnp.float32),
            pltpu.SemaphoreType.DMA((2,)),
            pltpu.SemaphoreType.DMA((2,)),
            pltpu.SemaphoreType.DMA((2,)),
            pltpu.SemaphoreType.DMA((2,)),
            pltpu.SemaphoreType.DMA((2,)),
        ],
        compiler_params=pltpu.CompilerParams(needs_layout_passes=False,
                                             use_tc_tiling_on_sc=False),
    )
    return f(src, dst, ee, a_src, a_dst, t2, edge_attr)


def _wnew_body(src_h, dst_h, us_h, ud_h, wet_h, wnew_h,
               src_c, dst_c, usr, udr, wet_c, sem, sem2):
    cid = lax.axis_index("c")
    sid = lax.axis_index("s")
    wid = cid * _NS + sid

    def chunk(k, carry):
        base = wid * _EPW + k * _C
        pltpu.sync_copy(src_h.at[pl.ds(base, _C)], src_c)
        pltpu.sync_copy(dst_h.at[pl.ds(base, _C)], dst_c)
        pltpu.sync_copy(wet_h.at[pl.ds(base, _C), :], wet_c)
        ca = pltpu.async_copy(us_h.at[src_c], usr, sem)
        cb = pltpu.async_copy(ud_h.at[dst_c], udr, sem2)
        ca.wait()
        cb.wait()
        for e in range(_C):
            wet_c[e, :] = wet_c[e, :] + usr[e, :] + udr[e, :]
        pltpu.sync_copy(wet_c, wnew_h.at[pl.ds(base, _C), :])
        return carry

    lax.fori_loop(0, _NCHUNK, chunk, 0)


def _wnew(src, dst, u_src, u_dst, w_et):
    mesh = plsc.VectorSubcoreMesh(core_axis_name="c", subcore_axis_name="s")
    f = pl.kernel(
        _wnew_body,
        out_type=jax.ShapeDtypeStruct((E, EA), jnp.float32),
        mesh=mesh,
        scratch_types=[
            pltpu.VMEM((_C,), jnp.int32),
            pltpu.VMEM((_C,), jnp.int32),
            pltpu.VMEM((_C, EA), jnp.float32),
            pltpu.VMEM((_C, EA), jnp.float32),
            pltpu.VMEM((_C, EA), jnp.float32),
            pltpu.SemaphoreType.DMA,
            pltpu.SemaphoreType.DMA,
        ],
        compiler_params=pltpu.CompilerParams(needs_layout_passes=False,
                                             use_tc_tiling_on_sc=False),
    )
    return f(src, dst, u_src, u_dst, w_et)


def kernel(x, edge_index, edge_attr, W_node, W_attn, W_to_node, W_edge):
    src = edge_index[0].astype(jnp.int32)
    dst = edge_index[1].astype(jnp.int32)
    Wa = jnp.concatenate([W_attn[:D], W_attn[D:2 * D]], axis=1)
    z, t2, a2 = _node_mms(x, W_node, W_to_node[:D], Wa)
    ee, w_et = _edge_mms(edge_attr, W_attn[2 * D:], W_edge[2 * D:])
    a_src = a2[:, 0]
    a_dst = a2[:, 1]

    hw_p, sp_p = _edge_pass(src, dst, ee[:, 0], a_src, a_dst, t2, edge_attr)

    h_new, u_src, u_dst = _combine(hw_p, sp_p, z,
                                   W_to_node[D:], W_edge[:D], W_edge[D:2 * D])

    w_new = _wnew(src, dst, u_src, u_dst, w_et)
    return h_new, w_new


# speedup vs baseline: 9.5226x; 1.1972x over previous
import functools

import jax
import jax.numpy as jnp
from jax import lax
from jax.experimental import pallas as pl
from jax.experimental.pallas import tpu as pltpu
from jax.experimental.pallas import tpu_sc as plsc

N = 10000
E = 320000
D = 128
EA = 16

_NBLK = 2000
_EBLK = 8000

_NC = 2
_NS = 16
_NW = _NC * _NS
_EPW = E // _NW
_C = 80
_NCHUNK = _EPW // _C
_NSH = 10240
_NPS = _NSH // _NS

_SP = 32


def _node_mm_body(x_ref, wn_ref, w1_ref, wa_ref, z_ref, t_ref, a_ref):
    z = jnp.dot(x_ref[...], wn_ref[...], preferred_element_type=jnp.float32)
    z_ref[...] = z
    t = jnp.dot(z, w1_ref[...], preferred_element_type=jnp.float32)
    t_ref[0] = t[:, :D // 2]
    t_ref[1] = t[:, D // 2:]
    a_ref[...] = jnp.dot(z, wa_ref[...], preferred_element_type=jnp.float32)


def _node_mms(x, W_node, W1, Wa):
    return pl.pallas_call(
        _node_mm_body,
        grid=(N // _NBLK,),
        in_specs=[
            pl.BlockSpec((_NBLK, D), lambda i: (i, 0)),
            pl.BlockSpec((D, D), lambda i: (0, 0)),
            pl.BlockSpec((D, D), lambda i: (0, 0)),
            pl.BlockSpec((D, 2), lambda i: (0, 0)),
        ],
        out_specs=[
            pl.BlockSpec((_NBLK, D), lambda i: (i, 0)),
            pl.BlockSpec((_NC, _NBLK, D // 2), lambda i: (0, i, 0)),
            pl.BlockSpec((_NBLK, 2), lambda i: (i, 0)),
        ],
        out_shape=[
            jax.ShapeDtypeStruct((N, D), jnp.float32),
            jax.ShapeDtypeStruct((_NC, N, D // 2), jnp.float32),
            jax.ShapeDtypeStruct((N, 2), jnp.float32),
        ],
    )(x, W_node, W1, Wa)


def _edge_mm_body(ea_ref, wa3_ref, we3_ref, ee_ref, wet_ref):
    ea = ea_ref[...]
    ee_ref[...] = jnp.dot(ea, wa3_ref[...], preferred_element_type=jnp.float32)
    wet_ref[...] = jnp.dot(ea, we3_ref[...], preferred_element_type=jnp.float32)


def _edge_mms(edge_attr, wa3, We3):
    return pl.pallas_call(
        _edge_mm_body,
        grid=(E // _EBLK,),
        in_specs=[
            pl.BlockSpec((_EBLK, EA), lambda i: (i, 0)),
            pl.BlockSpec((EA, 1), lambda i: (0, 0)),
            pl.BlockSpec((EA, EA), lambda i: (0, 0)),
        ],
        out_specs=[
            pl.BlockSpec((_EBLK, 1), lambda i: (i, 0)),
            pl.BlockSpec((_EBLK, EA), lambda i: (i, 0)),
        ],
        out_shape=[
            jax.ShapeDtypeStruct((E, 1), jnp.float32),
            jax.ShapeDtypeStruct((E, EA), jnp.float32),
        ],
    )(edge_attr, wa3, We3)


def _combine_body(hw_ref, sp_ref, z_ref, w2_ref, we1_ref, we2_ref,
                  h_ref, us_ref, ud_ref):
    hw = jnp.concatenate([hw_ref[0], hw_ref[1]], axis=-1)
    sp = sp_ref[0] + sp_ref[1]
    s16 = sp[:, 0:EA]
    den = sp[:, EA]
    h_agg = hw + jnp.dot(s16, w2_ref[...], preferred_element_type=jnp.float32)
    h_agg = h_agg / jnp.where(den > 0, den, 1.0)[:, None]
    h = jnp.where((den > 0)[:, None], h_agg, z_ref[...])
    h_ref[...] = h
    us_ref[...] = jnp.dot(h, we1_ref[...], preferred_element_type=jnp.float32)
    ud_ref[...] = jnp.dot(h, we2_ref[...], preferred_element_type=jnp.float32)


def _combine(hw_p, sp_p, z, W2, We1, We2):
    return pl.pallas_call(
        _combine_body,
        grid=(N // _NBLK,),
        in_specs=[
            pl.BlockSpec((_NC, _NBLK, D // 2), lambda i: (0, i, 0)),
            pl.BlockSpec((_NC, _NBLK, _SP), lambda i: (0, i, 0)),
            pl.BlockSpec((_NBLK, D), lambda i: (i, 0)),
            pl.BlockSpec((EA, D), lambda i: (0, 0)),
            pl.BlockSpec((D, EA), lambda i: (0, 0)),
            pl.BlockSpec((D, EA), lambda i: (0, 0)),
        ],
        out_specs=[
            pl.BlockSpec((_NBLK, D), lambda i: (i, 0)),
            pl.BlockSpec((_NBLK, EA), lambda i: (i, 0)),
            pl.BlockSpec((_NBLK, EA), lambda i: (i, 0)),
        ],
        out_shape=[
            jax.ShapeDtypeStruct((N, D), jnp.float32),
            jax.ShapeDtypeStruct((N, EA), jnp.float32),
            jax.ShapeDtypeStruct((N, EA), jnp.float32),
        ],
    )(hw_p, sp_p, z, W2, We1, We2)


_EPT = E // _NS
_C = 80
_NCH = _EPT // _C
_DH = D // 2


def _edge_pass_body(src_h, dst_h, ee_h, asrc_h, adst_h, tn2_h, ea_h,
                    hwp_h, spp_h,
                    asrc_v, adst_v, srcb, dstb, sdst, eeb, eab, gbuf, sbuf,
                    spb, hw_sh, sp_sh, gsem, rsem, psem, lsem, ssem):
    cid = lax.axis_index("c")
    sid = lax.axis_index("s")

    pltpu.sync_copy(asrc_h, asrc_v)
    pltpu.sync_copy(adst_h, adst_v)

    zz = jnp.zeros((16,), jnp.float32)
    for r in range(_C):
        for q in range(_DH // 16):
            sbuf[0, r, pl.ds(q * 16, 16)] = zz
        for q in range(_SP // 16):
            spb[0, r, pl.ds(q * 16, 16)] = zz
    for k in range(_NPS // _C):
        pltpu.sync_copy(sbuf.at[0], hw_sh.at[pl.ds(sid * _NPS + k * _C, _C), :])
        pltpu.sync_copy(spb.at[0], sp_sh.at[pl.ds(sid * _NPS + k * _C, _C), :])
    plsc.subcore_barrier()

    ebase = sid * _EPT

    def start_src(k, b):
        pltpu.make_async_copy(src_h.at[pl.ds(ebase + k * _C, _C)],
                              srcb.at[b], ssem.at[b]).start()

    def drain_src(b):
        pltpu.make_async_copy(src_h.at[pl.ds(0, _C)], srcb.at[b],
                              ssem.at[b]).wait()

    def start_lin(k, b):
        pltpu.make_async_copy(dst_h.at[pl.ds(ebase + k * _C, _C)],
                              dstb.at[b], lsem.at[b]).start()
        pltpu.make_async_copy(ee_h.at[pl.ds(ebase + k * _C, _C)],
                              eeb.at[b], lsem.at[b]).start()
        pltpu.make_async_copy(ea_h.at[pl.ds(ebase + k * _C, _C), :],
                              eab.at[b], lsem.at[b]).start()

    def drain_lin(b):
        pltpu.make_async_copy(dst_h.at[pl.ds(0, _C)], dstb.at[b],
                              lsem.at[b]).wait()
        pltpu.make_async_copy(ee_h.at[pl.ds(0, _C)], eeb.at[b],
                              lsem.at[b]).wait()
        pltpu.make_async_copy(ea_h.at[pl.ds(0, _C), :], eab.at[b],
                              lsem.at[b]).wait()

    def start_gather(bsrc, b):
        pltpu.make_async_copy(tn2_h.at[cid].at[srcb.at[bsrc]],
                              gbuf.at[b], gsem.at[b]).start()

    def drain_gather(b):
        pltpu.make_async_copy(tn2_h.at[cid].at[srcb.at[0]],
                              gbuf.at[b], gsem.at[b]).wait()

    def start_scat(b):
        pltpu.make_async_copy(sbuf.at[b], hw_sh.at[sdst.at[b]],
                              rsem.at[b]).start(add=True)

    def drain_scat(b):
        pltpu.make_async_copy(sbuf.at[b], hw_sh.at[sdst.at[0]],
                              rsem.at[b]).wait()

    def start_spscat(b):
        pltpu.make_async_copy(spb.at[b], sp_sh.at[sdst.at[b]],
                              psem.at[b]).start(add=True)

    def drain_spscat(b):
        pltpu.make_async_copy(spb.at[b], sp_sh.at[sdst.at[0]],
                              psem.at[b]).wait()

    def compute(b):
        for g in range(_C // 16):
            sv = srcb[b, pl.ds(g * 16, 16)]
            dv = dstb[b, pl.ds(g * 16, 16)]
            sdst[b, pl.ds(g * 16, 16)] = dv
            av = plsc.load_gather(asrc_v, [sv])
            bv = plsc.load_gather(adst_v, [dv])
            ev = av + bv + eeb[b, pl.ds(g * 16, 16)]
            ev = jnp.where(ev >= 0.0, ev, 0.1 * ev)
            pv = jnp.exp(ev)
            for l in range(16):
                e = g * 16 + l
                pvb = jnp.full((16,), pv[l], jnp.float32)
                for q in range(_DH // 16):
                    sbuf[b, e, pl.ds(q * 16, 16)] = (
                        gbuf[b, e, pl.ds(q * 16, 16)] * pvb)
                spb[b, e, pl.ds(0, 16)] = eab[b, e, :] * pvb
                spb[b, e, pl.ds(16, 16)] = pvb

    def half(k, b):
        @pl.when(k >= 2)
        def _():
            drain_scat(b)

        @pl.when((k >= 2) & (lax.rem(k, 2) == cid))
        def _():
            drain_spscat(b)

        @pl.when(k + 1 < _NCH)
        def _():
            start_lin(k + 1, 1 - b)

        drain_gather(b)

        @pl.when(k + 1 < _NCH)
        def _():
            drain_src(1 - b)
            start_gather(1 - b, 1 - b)

        drain_lin(b)
        compute(b)
        start_scat(b)

        @pl.when(lax.rem(k, 2) == cid)
        def _():
            start_spscat(b)

        @pl.when(k + 2 < _NCH)
        def _():
            start_src(k + 2, b)

    start_src(0, 0)
    start_src(1, 1)
    start_lin(0, 0)
    drain_src(0)
    start_gather(0, 0)

    def pair(t, carry):
        half(2 * t, 0)
        half(2 * t + 1, 1)
        return carry

    lax.fori_loop(0, _NCH // 2, pair, 0)

    drain_scat((_NCH - 1) % 2)
    drain_scat((_NCH - 2) % 2)

    @pl.when(cid == 0)
    def _():
        drain_spscat(0)

    @pl.when(cid == 1)
    def _():
        drain_spscat(1)

    plsc.subcore_barrier()

    pltpu.sync_copy(hw_sh.at[pl.ds(sid * _NPS, _NPS), :],
                    hwp_h.at[cid, pl.ds(sid * _NPS, _NPS), :])
    pltpu.sync_copy(sp_sh.at[pl.ds(sid * _NPS, _NPS), :],
                    spp_h.at[cid, pl.ds(sid * _NPS, _NPS), :])


def _edge_pass(src, dst, ee, a_src, a_dst, t2, edge_attr):
    mesh = plsc.VectorSubcoreMesh(core_axis_name="c", subcore_axis_name="s")
    f = pl.kernel(
        _edge_pass_body,
        out_type=[
            jax.ShapeDtypeStruct((_NC, _NSH, _DH), jnp.float32),
            jax.ShapeDtypeStruct((_NC, _NSH, _SP), jnp.float32),
        ],
        mesh=mesh,
        scratch_types=[
            pltpu.VMEM((N,), jnp.float32),
            pltpu.VMEM((N,), jnp.float32),
            pltpu.VMEM((2, _C), jnp.int32),
            pltpu.VMEM((2, _C), jnp.int32),
            pltpu.VMEM((2, _C), jnp.int32),
            pltpu.VMEM((2, _C), jnp.float32),
            pltpu.VMEM((2, _C, EA), jnp.float32),
            pltpu.VMEM((2, _C, _DH), jnp.float32),
            pltpu.VMEM((2, _C, _DH), jnp.float32),
            pltpu.VMEM((2, _C, _SP), jnp.float32),
            pltpu.VMEM_SHARED((_NSH, _DH), jnp.float32),
            pltpu.VMEM_SHARED((_NSH, _SP), jnp.float32),
            pltpu.SemaphoreType.DMA((2,)),
            pltpu.SemaphoreType.DMA((2,)),
            pltpu.SemaphoreType.DMA((2,)),
            pltpu.SemaphoreType.DMA((2,)),
            pltpu.SemaphoreType.DMA((2,)),
        ],
        compiler_params=pltpu.CompilerParams(needs_layout_passes=False,
                                             use_tc_tiling_on_sc=False),
    )
    return f(src, dst, ee, a_src, a_dst, t2, edge_attr)


_CW = 400
_NSUBW = _CW // 80
_NCHW = _EPT // _CW


def _wnew_body(src_h, dst_h, us_h, ud_h, wet_h, wnew_h,
               srcb, dstb, usr, udr, wetb, outb,
               gsem, lsem, ssem, wsem):
    sid = lax.axis_index("s")
    cid = lax.axis_index("c")
    ebase = (sid * _NC + cid) * (_EPT // _NC)
    nch = _NCHW // _NC

    def start_src(k, b):
        pltpu.make_async_copy(src_h.at[pl.ds(ebase + k * _CW, _CW)],
                              srcb.at[b], ssem.at[b]).start()
        pltpu.make_async_copy(dst_h.at[pl.ds(ebase + k * _CW, _CW)],
                              dstb.at[b], ssem.at[b]).start()

    def drain_src(b):
        pltpu.make_async_copy(src_h.at[pl.ds(0, _CW)], srcb.at[b],
                              ssem.at[b]).wait()
        pltpu.make_async_copy(dst_h.at[pl.ds(0, _CW)], dstb.at[b],
                              ssem.at[b]).wait()

    def start_lin(k, b):
        pltpu.make_async_copy(wet_h.at[pl.ds(ebase + k * _CW, _CW), :],
                              wetb.at[b], lsem.at[b]).start()

    def drain_lin(b):
        pltpu.make_async_copy(wet_h.at[pl.ds(0, _CW), :], wetb.at[b],
                              lsem.at[b]).wait()

    def start_gather(b):
        for j in range(_NSUBW):
            pltpu.make_async_copy(
                us_h.at[srcb.at[b, pl.ds(j * 80, 80)]],
                usr.at[b, pl.ds(j * 80, 80), :], gsem.at[b]).start()
            pltpu.make_async_copy(
                ud_h.at[dstb.at[b, pl.ds(j * 80, 80)]],
                udr.at[b, pl.ds(j * 80, 80), :], gsem.at[b]).start()

    def drain_gather(b):
        for j in range(_NSUBW):
            pltpu.make_async_copy(
                us_h.at[srcb.at[0, pl.ds(0, 80)]],
                usr.at[b, pl.ds(j * 80, 80), :], gsem.at[b]).wait()
            pltpu.make_async_copy(
                ud_h.at[dstb.at[0, pl.ds(0, 80)]],
                udr.at[b, pl.ds(j * 80, 80), :], gsem.at[b]).wait()

    def start_store(k, b):
        pltpu.make_async_copy(outb.at[b],
                              wnew_h.at[pl.ds(ebase + k * _CW, _CW), :],
                              wsem.at[b]).start()

    def drain_store(b):
        pltpu.make_async_copy(outb.at[b],
                              wnew_h.at[pl.ds(0, _CW), :],
                              wsem.at[b]).wait()

    def compute(b):
        for e in range(_CW):
            outb[b, e, :] = usr[b, e, :] + udr[b, e, :] + wetb[b, e, :]

    def half(k, b):
        @pl.when(k >= 2)
        def _():
            drain_store(b)

        @pl.when(k + 1 < nch)
        def _():
            start_lin(k + 1, 1 - b)

        drain_gather(b)

        @pl.when(k + 1 < nch)
        def _():
            drain_src(1 - b)
            start_gather(1 - b)

        drain_lin(b)
        compute(b)
        start_store(k, b)

        @pl.when(k + 2 < nch)
        def _():
            start_src(k + 2, b)

    start_src(0, 0)
    start_src(1, 1)
    start_lin(0, 0)
    drain_src(0)
    start_gather(0)

    def pair(t, carry):
        half(2 * t, 0)
        half(2 * t + 1, 1)
        return carry

    lax.fori_loop(0, nch // 2, pair, 0)
    if nch % 2:
        half(nch - 1, 0)

    drain_store((nch - 1) % 2)
    drain_store((nch - 2) % 2)


def _wnew(src, dst, u_src, u_dst, w_et):
    mesh = plsc.VectorSubcoreMesh(core_axis_name="c", subcore_axis_name="s")
    f = pl.kernel(
        _wnew_body,
        out_type=jax.ShapeDtypeStruct((E, EA), jnp.float32),
        mesh=mesh,
        scratch_types=[
            pltpu.VMEM((2, _CW), jnp.int32),
            pltpu.VMEM((2, _CW), jnp.int32),
            pltpu.VMEM((2, _CW, EA), jnp.float32),
            pltpu.VMEM((2, _CW, EA), jnp.float32),
            pltpu.VMEM((2, _CW, EA), jnp.float32),
            pltpu.VMEM((2, _CW, EA), jnp.float32),
            pltpu.SemaphoreType.DMA((2,)),
            pltpu.SemaphoreType.DMA((2,)),
            pltpu.SemaphoreType.DMA((2,)),
            pltpu.SemaphoreType.DMA((2,)),
        ],
        compiler_params=pltpu.CompilerParams(needs_layout_passes=False,
                                             use_tc_tiling_on_sc=False),
    )
    return f(src, dst, u_src, u_dst, w_et)


def kernel(x, edge_index, edge_attr, W_node, W_attn, W_to_node, W_edge):
    src = edge_index[0].astype(jnp.int32)
    dst = edge_index[1].astype(jnp.int32)
    Wa = jnp.concatenate([W_attn[:D], W_attn[D:2 * D]], axis=1)
    z, t2, a2 = _node_mms(x, W_node, W_to_node[:D], Wa)
    ee, w_et = _edge_mms(edge_attr, W_attn[2 * D:], W_edge[2 * D:])
    a_src = a2[:, 0]
    a_dst = a2[:, 1]

    hw_p, sp_p = _edge_pass(src, dst, ee[:, 0], a_src, a_dst, t2, edge_attr)

    h_new, u_src, u_dst = _combine(hw_p, sp_p, z,
                                   W_to_node[D:], W_edge[:D], W_edge[D:2 * D])

    w_new = _wnew(src, dst, u_src, u_dst, w_et)
    return h_new, w_new


# speedup vs baseline: 9.7340x; 1.0222x over previous
import functools

import jax
import jax.numpy as jnp
from jax import lax
from jax.experimental import pallas as pl
from jax.experimental.pallas import tpu as pltpu
from jax.experimental.pallas import tpu_sc as plsc

N = 10000
E = 320000
D = 128
EA = 16

_NBLK = 2000
_EBLK = 8000

_NC = 2
_NS = 16
_NW = _NC * _NS
_EPW = E // _NW
_C = 80
_NCHUNK = _EPW // _C
_NSH = 10240
_NPS = _NSH // _NS

_SP = 32


def _node_mm_body(x_ref, wn_ref, w1_ref, wa_ref, z_ref, t_ref, a_ref):
    z = jnp.dot(x_ref[...], wn_ref[...], preferred_element_type=jnp.float32)
    z_ref[...] = z
    t = jnp.dot(z, w1_ref[...], preferred_element_type=jnp.float32)
    t_ref[0] = t[:, :D // 2]
    t_ref[1] = t[:, D // 2:]
    a_ref[...] = jnp.dot(z, wa_ref[...], preferred_element_type=jnp.float32)


def _node_mms(x, W_node, W1, Wa):
    return pl.pallas_call(
        _node_mm_body,
        grid=(N // _NBLK,),
        in_specs=[
            pl.BlockSpec((_NBLK, D), lambda i: (i, 0)),
            pl.BlockSpec((D, D), lambda i: (0, 0)),
            pl.BlockSpec((D, D), lambda i: (0, 0)),
            pl.BlockSpec((D, 2), lambda i: (0, 0)),
        ],
        out_specs=[
            pl.BlockSpec((_NBLK, D), lambda i: (i, 0)),
            pl.BlockSpec((_NC, _NBLK, D // 2), lambda i: (0, i, 0)),
            pl.BlockSpec((_NBLK, 2), lambda i: (i, 0)),
        ],
        out_shape=[
            jax.ShapeDtypeStruct((N, D), jnp.float32),
            jax.ShapeDtypeStruct((_NC, N, D // 2), jnp.float32),
            jax.ShapeDtypeStruct((N, 2), jnp.float32),
        ],
    )(x, W_node, W1, Wa)


def _edge_mm_body(ea_ref, we3_ref, wet_ref):
    ea = ea_ref[...]
    wet_ref[...] = jnp.dot(ea, we3_ref[...], preferred_element_type=jnp.float32)


def _edge_mms(edge_attr, We3):
    return pl.pallas_call(
        _edge_mm_body,
        grid=(E // _EBLK,),
        in_specs=[
            pl.BlockSpec((_EBLK, EA), lambda i: (i, 0)),
            pl.BlockSpec((EA, EA), lambda i: (0, 0)),
        ],
        out_specs=pl.BlockSpec((_EBLK, EA), lambda i: (i, 0)),
        out_shape=jax.ShapeDtypeStruct((E, EA), jnp.float32),
    )(edge_attr, We3)


def _combine_body(hw_ref, sp_ref, z_ref, w2_ref, we1_ref, we2_ref,
                  h_ref, us_ref, ud_ref):
    hw = jnp.concatenate([hw_ref[0], hw_ref[1]], axis=-1)
    sp = sp_ref[0] + sp_ref[1]
    s16 = sp[:, 0:EA]
    den = sp[:, EA]
    h_agg = hw + jnp.dot(s16, w2_ref[...], preferred_element_type=jnp.float32)
    h_agg = h_agg / jnp.where(den > 0, den, 1.0)[:, None]
    h = jnp.where((den > 0)[:, None], h_agg, z_ref[...])
    h_ref[...] = h
    us_ref[...] = jnp.dot(h, we1_ref[...], preferred_element_type=jnp.float32)
    ud_ref[...] = jnp.dot(h, we2_ref[...], preferred_element_type=jnp.float32)


def _combine(hw_p, sp_p, z, W2, We1, We2):
    return pl.pallas_call(
        _combine_body,
        grid=(N // _NBLK,),
        in_specs=[
            pl.BlockSpec((_NC, _NBLK, D // 2), lambda i: (0, i, 0)),
            pl.BlockSpec((_NC, _NBLK, _SP), lambda i: (0, i, 0)),
            pl.BlockSpec((_NBLK, D), lambda i: (i, 0)),
            pl.BlockSpec((EA, D), lambda i: (0, 0)),
            pl.BlockSpec((D, EA), lambda i: (0, 0)),
            pl.BlockSpec((D, EA), lambda i: (0, 0)),
        ],
        out_specs=[
            pl.BlockSpec((_NBLK, D), lambda i: (i, 0)),
            pl.BlockSpec((_NBLK, EA), lambda i: (i, 0)),
            pl.BlockSpec((_NBLK, EA), lambda i: (i, 0)),
        ],
        out_shape=[
            jax.ShapeDtypeStruct((N, D), jnp.float32),
            jax.ShapeDtypeStruct((N, EA), jnp.float32),
            jax.ShapeDtypeStruct((N, EA), jnp.float32),
        ],
    )(hw_p, sp_p, z, W2, We1, We2)


_EPT = E // _NS
_C = 80
_NCH = _EPT // _C
_DH = D // 2


def _edge_pass_body(ei_h, a2_h, wa3_h, tn2_h, ea_h,
                    hwp_h, spp_h,
                    a2_v, wa3_v, srcb, dstb, sdst, eab, gbuf, sbuf,
                    spb, hw_sh, sp_sh, gsem, rsem, psem, lsem, ssem):
    cid = lax.axis_index("c")
    sid = lax.axis_index("s")

    pltpu.sync_copy(a2_h, a2_v)
    pltpu.sync_copy(wa3_h, wa3_v)

    zz = jnp.zeros((16,), jnp.float32)
    for r in range(_C):
        for q in range(_DH // 16):
            sbuf[0, r, pl.ds(q * 16, 16)] = zz
        for q in range(_SP // 16):
            spb[0, r, pl.ds(q * 16, 16)] = zz
    for k in range(_NPS // _C):
        pltpu.sync_copy(sbuf.at[0], hw_sh.at[pl.ds(sid * _NPS + k * _C, _C), :])
        pltpu.sync_copy(spb.at[0], sp_sh.at[pl.ds(sid * _NPS + k * _C, _C), :])
    plsc.subcore_barrier()

    ebase = sid * _EPT

    def start_src(k, b):
        pltpu.make_async_copy(ei_h.at[0, pl.ds(ebase + k * _C, _C)],
                              srcb.at[b], ssem.at[b]).start()

    def drain_src(b):
        pltpu.make_async_copy(ei_h.at[0, pl.ds(0, _C)], srcb.at[b],
                              ssem.at[b]).wait()

    def start_lin(k, b):
        pltpu.make_async_copy(ei_h.at[1, pl.ds(ebase + k * _C, _C)],
                              dstb.at[b], lsem.at[b]).start()
        pltpu.make_async_copy(ea_h.at[pl.ds(ebase + k * _C, _C), :],
                              eab.at[b], lsem.at[b]).start()

    def drain_lin(b):
        pltpu.make_async_copy(ei_h.at[1, pl.ds(0, _C)], dstb.at[b],
                              lsem.at[b]).wait()
        pltpu.make_async_copy(ea_h.at[pl.ds(0, _C), :], eab.at[b],
                              lsem.at[b]).wait()

    def start_gather(bsrc, b):
        pltpu.make_async_copy(tn2_h.at[cid].at[srcb.at[bsrc]],
                              gbuf.at[b], gsem.at[b]).start()

    def drain_gather(b):
        pltpu.make_async_copy(tn2_h.at[cid].at[srcb.at[0]],
                              gbuf.at[b], gsem.at[b]).wait()

    def start_scat(b):
        pltpu.make_async_copy(sbuf.at[b], hw_sh.at[sdst.at[b]],
                              rsem.at[b]).start(add=True)

    def drain_scat(b):
        pltpu.make_async_copy(sbuf.at[b], hw_sh.at[sdst.at[0]],
                              rsem.at[b]).wait()

    def start_spscat(b):
        pltpu.make_async_copy(spb.at[b], sp_sh.at[sdst.at[b]],
                              psem.at[b]).start(add=True)

    def drain_spscat(b):
        pltpu.make_async_copy(spb.at[b], sp_sh.at[sdst.at[0]],
                              psem.at[b]).wait()

    def compute(b):
        w3 = wa3_v[...]
        w3s = [jnp.full((16,), w3[f], jnp.float32) for f in range(EA)]
        for g in range(_C // 16):
            sv = srcb[b, pl.ds(g * 16, 16)]
            dv = dstb[b, pl.ds(g * 16, 16)]
            sdst[b, pl.ds(g * 16, 16)] = dv
            av = plsc.load_gather(a2_v, [sv * 2])
            bv = plsc.load_gather(a2_v, [dv * 2 + 1])
            eids = lax.iota(jnp.int32, 16) + (g * 16)
            ee = jnp.zeros((16,), jnp.float32)
            for f in range(EA):
                fidx = jnp.full((16,), f, jnp.int32)
                col = plsc.load_gather(eab.at[b], [eids, fidx])
                ee = ee + col * w3s[f]
            ev = av + bv + ee
            ev = jnp.where(ev >= 0.0, ev, 0.1 * ev)
            pv = jnp.exp(ev)
            for l in range(16):
                e = g * 16 + l
                pvb = jnp.full((16,), pv[l], jnp.float32)
                for q in range(_DH // 16):
                    sbuf[b, e, pl.ds(q * 16, 16)] = (
                        gbuf[b, e, pl.ds(q * 16, 16)] * pvb)
                spb[b, e, pl.ds(0, 16)] = eab[b, e, :] * pvb
                spb[b, e, pl.ds(16, 16)] = pvb

    def half(k, b):
        @pl.when(k >= 2)
        def _():
            drain_scat(b)

        @pl.when((k >= 2) & (lax.rem(k, 2) == cid))
        def _():
            drain_spscat(b)

        @pl.when(k + 1 < _NCH)
        def _():
            start_lin(k + 1, 1 - b)

        drain_gather(b)

        @pl.when(k + 1 < _NCH)
        def _():
            drain_src(1 - b)
            start_gather(1 - b, 1 - b)

        drain_lin(b)
        compute(b)
        start_scat(b)

        @pl.when(lax.rem(k, 2) == cid)
        def _():
            start_spscat(b)

        @pl.when(k + 2 < _NCH)
        def _():
            start_src(k + 2, b)

    start_src(0, 0)
    start_src(1, 1)
    start_lin(0, 0)
    drain_src(0)
    start_gather(0, 0)

    def pair(t, carry):
        half(2 * t, 0)
        half(2 * t + 1, 1)
        return carry

    lax.fori_loop(0, _NCH // 2, pair, 0)

    drain_scat((_NCH - 1) % 2)
    drain_scat((_NCH - 2) % 2)

    @pl.when(cid == 0)
    def _():
        drain_spscat(0)

    @pl.when(cid == 1)
    def _():
        drain_spscat(1)

    plsc.subcore_barrier()

    pltpu.sync_copy(hw_sh.at[pl.ds(sid * _NPS, _NPS), :],
                    hwp_h.at[cid, pl.ds(sid * _NPS, _NPS), :])
    pltpu.sync_copy(sp_sh.at[pl.ds(sid * _NPS, _NPS), :],
                    spp_h.at[cid, pl.ds(sid * _NPS, _NPS), :])


def _edge_pass(ei, a2, wa3, t2, edge_attr):
    mesh = plsc.VectorSubcoreMesh(core_axis_name="c", subcore_axis_name="s")
    f = pl.kernel(
        _edge_pass_body,
        out_type=[
            jax.ShapeDtypeStruct((_NC, _NSH, _DH), jnp.float32),
            jax.ShapeDtypeStruct((_NC, _NSH, _SP), jnp.float32),
        ],
        mesh=mesh,
        scratch_types=[
            pltpu.VMEM((2 * N,), jnp.float32),
            pltpu.VMEM((EA,), jnp.float32),
            pltpu.VMEM((2, _C), jnp.int32),
            pltpu.VMEM((2, _C), jnp.int32),
            pltpu.VMEM((2, _C), jnp.int32),
            pltpu.VMEM((2, _C, EA), jnp.float32),
            pltpu.VMEM((2, _C, _DH), jnp.float32),
            pltpu.VMEM((2, _C, _DH), jnp.float32),
            pltpu.VMEM((2, _C, _SP), jnp.float32),
            pltpu.VMEM_SHARED((_NSH, _DH), jnp.float32),
            pltpu.VMEM_SHARED((_NSH, _SP), jnp.float32),
            pltpu.SemaphoreType.DMA((2,)),
            pltpu.SemaphoreType.DMA((2,)),
            pltpu.SemaphoreType.DMA((2,)),
            pltpu.SemaphoreType.DMA((2,)),
            pltpu.SemaphoreType.DMA((2,)),
        ],
        compiler_params=pltpu.CompilerParams(needs_layout_passes=False,
                                             use_tc_tiling_on_sc=False),
    )
    return f(ei, a2, wa3, t2, edge_attr)


_CW = 400
_NSUBW = _CW // 80
_NCHW = _EPT // _CW


def _wnew_body(ei_h, us_h, ud_h, wet_h, wnew_h,
               srcb, dstb, usr, udr, wetb, outb,
               gsem, lsem, ssem, wsem):
    sid = lax.axis_index("s")
    cid = lax.axis_index("c")
    ebase = (sid * _NC + cid) * (_EPT // _NC)
    nch = _NCHW // _NC

    def start_src(k, b):
        pltpu.make_async_copy(ei_h.at[0, pl.ds(ebase + k * _CW, _CW)],
                              srcb.at[b], ssem.at[b]).start()
        pltpu.make_async_copy(ei_h.at[1, pl.ds(ebase + k * _CW, _CW)],
                              dstb.at[b], ssem.at[b]).start()

    def drain_src(b):
        pltpu.make_async_copy(ei_h.at[0, pl.ds(0, _CW)], srcb.at[b],
                              ssem.at[b]).wait()
        pltpu.make_async_copy(ei_h.at[1, pl.ds(0, _CW)], dstb.at[b],
                              ssem.at[b]).wait()

    def start_lin(k, b):
        pltpu.make_async_copy(wet_h.at[pl.ds(ebase + k * _CW, _CW), :],
                              wetb.at[b], lsem.at[b]).start()

    def drain_lin(b):
        pltpu.make_async_copy(wet_h.at[pl.ds(0, _CW), :], wetb.at[b],
                              lsem.at[b]).wait()

    def start_gather(b):
        for j in range(_NSUBW):
            pltpu.make_async_copy(
                us_h.at[srcb.at[b, pl.ds(j * 80, 80)]],
                usr.at[b, pl.ds(j * 80, 80), :], gsem.at[b]).start()
            pltpu.make_async_copy(
                ud_h.at[dstb.at[b, pl.ds(j * 80, 80)]],
                udr.at[b, pl.ds(j * 80, 80), :], gsem.at[b]).start()

    def drain_gather(b):
        for j in range(_NSUBW):
            pltpu.make_async_copy(
                us_h.at[srcb.at[0, pl.ds(0, 80)]],
                usr.at[b, pl.ds(j * 80, 80), :], gsem.at[b]).wait()
            pltpu.make_async_copy(
                ud_h.at[dstb.at[0, pl.ds(0, 80)]],
                udr.at[b, pl.ds(j * 80, 80), :], gsem.at[b]).wait()

    def start_store(k, b):
        pltpu.make_async_copy(outb.at[b],
                              wnew_h.at[pl.ds(ebase + k * _CW, _CW), :],
                              wsem.at[b]).start()

    def drain_store(b):
        pltpu.make_async_copy(outb.at[b],
                              wnew_h.at[pl.ds(0, _CW), :],
                              wsem.at[b]).wait()

    def compute(b):
        for e in range(_CW):
            outb[b, e, :] = usr[b, e, :] + udr[b, e, :] + wetb[b, e, :]

    def half(k, b):
        @pl.when(k >= 2)
        def _():
            drain_store(b)

        @pl.when(k + 1 < nch)
        def _():
            start_lin(k + 1, 1 - b)

        drain_gather(b)

        @pl.when(k + 1 < nch)
        def _():
            drain_src(1 - b)
            start_gather(1 - b)

        drain_lin(b)
        compute(b)
        start_store(k, b)

        @pl.when(k + 2 < nch)
        def _():
            start_src(k + 2, b)

    start_src(0, 0)
    start_src(1, 1)
    start_lin(0, 0)
    drain_src(0)
    start_gather(0)

    def pair(t, carry):
        half(2 * t, 0)
        half(2 * t + 1, 1)
        return carry

    lax.fori_loop(0, nch // 2, pair, 0)
    if nch % 2:
        half(nch - 1, 0)

    drain_store((nch - 1) % 2)
    drain_store((nch - 2) % 2)


def _wnew(ei, u_src, u_dst, w_et):
    mesh = plsc.VectorSubcoreMesh(core_axis_name="c", subcore_axis_name="s")
    f = pl.kernel(
        _wnew_body,
        out_type=jax.ShapeDtypeStruct((E, EA), jnp.float32),
        mesh=mesh,
        scratch_types=[
            pltpu.VMEM((2, _CW), jnp.int32),
            pltpu.VMEM((2, _CW), jnp.int32),
            pltpu.VMEM((2, _CW, EA), jnp.float32),
            pltpu.VMEM((2, _CW, EA), jnp.float32),
            pltpu.VMEM((2, _CW, EA), jnp.float32),
            pltpu.VMEM((2, _CW, EA), jnp.float32),
            pltpu.SemaphoreType.DMA((2,)),
            pltpu.SemaphoreType.DMA((2,)),
            pltpu.SemaphoreType.DMA((2,)),
            pltpu.SemaphoreType.DMA((2,)),
        ],
        compiler_params=pltpu.CompilerParams(needs_layout_passes=False,
                                             use_tc_tiling_on_sc=False),
    )
    return f(ei, u_src, u_dst, w_et)


def kernel(x, edge_index, edge_attr, W_node, W_attn, W_to_node, W_edge):
    ei = edge_index.astype(jnp.int32)
    Wa = jnp.concatenate([W_attn[:D], W_attn[D:2 * D]], axis=1)
    z, t2, a2 = _node_mms(x, W_node, W_to_node[:D], Wa)
    w_et = _edge_mms(edge_attr, W_edge[2 * D:])

    hw_p, sp_p = _edge_pass(ei, a2.reshape(2 * N), W_attn[2 * D:, 0], t2,
                            edge_attr)

    h_new, u_src, u_dst = _combine(hw_p, sp_p, z,
                                   W_to_node[D:], W_edge[:D], W_edge[D:2 * D])

    w_new = _wnew(ei, u_src, u_dst, w_et)
    return h_new, w_new


# speedup vs baseline: 9.7428x; 1.0009x over previous
import jax
import jax.numpy as jnp
from jax import lax
from jax.experimental import pallas as pl
from jax.experimental.pallas import tpu as pltpu
from jax.experimental.pallas import tpu_sc as plsc

N = 10000
E = 320000
D = 128
EA = 16

_NBLK = 2000
_EBLK = 8000

_NC = 2
_NS = 16
_NW = _NC * _NS
_EPW = E // _NW
_C = 80
_NCHUNK = _EPW // _C
_NSH = 10240
_NPS = _NSH // _NS

_SP = 32


def _node_mm_body(x_ref, wn_ref, w1_ref, wa_ref, z_ref, t_ref, a_ref):
    z = jnp.dot(x_ref[...], wn_ref[...], preferred_element_type=jnp.float32)
    z_ref[...] = z
    t = jnp.dot(z, w1_ref[...], preferred_element_type=jnp.float32)
    t_ref[0] = t[:, :D // 2]
    t_ref[1] = t[:, D // 2:]
    a_ref[...] = jnp.dot(z, wa_ref[...], preferred_element_type=jnp.float32)


def _node_mms(x, W_node, W1, Wa):
    return pl.pallas_call(
        _node_mm_body,
        grid=(N // _NBLK,),
        in_specs=[
            pl.BlockSpec((_NBLK, D), lambda i: (i, 0)),
            pl.BlockSpec((D, D), lambda i: (0, 0)),
            pl.BlockSpec((D, D), lambda i: (0, 0)),
            pl.BlockSpec((D, 2), lambda i: (0, 0)),
        ],
        out_specs=[
            pl.BlockSpec((_NBLK, D), lambda i: (i, 0)),
            pl.BlockSpec((_NC, _NBLK, D // 2), lambda i: (0, i, 0)),
            pl.BlockSpec((_NBLK, 2), lambda i: (i, 0)),
        ],
        out_shape=[
            jax.ShapeDtypeStruct((N, D), jnp.float32),
            jax.ShapeDtypeStruct((_NC, N, D // 2), jnp.float32),
            jax.ShapeDtypeStruct((N, 2), jnp.float32),
        ],
    )(x, W_node, W1, Wa)


def _edge_mm_body(ea_ref, we3_ref, wet_ref):
    ea = ea_ref[...]
    wet_ref[...] = jnp.dot(ea, we3_ref[...], preferred_element_type=jnp.float32)


def _edge_mms(edge_attr, We3):
    return pl.pallas_call(
        _edge_mm_body,
        grid=(E // _EBLK,),
        in_specs=[
            pl.BlockSpec((_EBLK, EA), lambda i: (i, 0)),
            pl.BlockSpec((EA, EA), lambda i: (0, 0)),
        ],
        out_specs=pl.BlockSpec((_EBLK, EA), lambda i: (i, 0)),
        out_shape=jax.ShapeDtypeStruct((E, EA), jnp.float32),
    )(edge_attr, We3)


def _combine_body(hw_ref, sp_ref, z_ref, w2_ref, we1_ref, we2_ref,
                  h_ref, us_ref, ud_ref):
    hw = jnp.concatenate([hw_ref[0], hw_ref[1]], axis=-1)
    sp = sp_ref[0] + sp_ref[1]
    s16 = sp[:, 0:EA]
    den = sp[:, EA]
    h_agg = hw + jnp.dot(s16, w2_ref[...], preferred_element_type=jnp.float32)
    h_agg = h_agg / jnp.where(den > 0, den, 1.0)[:, None]
    h = jnp.where((den > 0)[:, None], h_agg, z_ref[...])
    h_ref[...] = h
    us_ref[...] = jnp.dot(h, we1_ref[...], preferred_element_type=jnp.float32)
    ud_ref[...] = jnp.dot(h, we2_ref[...], preferred_element_type=jnp.float32)


def _combine(hw_p, sp_p, z, W2, We1, We2):
    return pl.pallas_call(
        _combine_body,
        grid=(N // _NBLK,),
        in_specs=[
            pl.BlockSpec((_NC, _NBLK, D // 2), lambda i: (0, i, 0)),
            pl.BlockSpec((_NC, _NBLK, _SP), lambda i: (0, i, 0)),
            pl.BlockSpec((_NBLK, D), lambda i: (i, 0)),
            pl.BlockSpec((EA, D), lambda i: (0, 0)),
            pl.BlockSpec((D, EA), lambda i: (0, 0)),
            pl.BlockSpec((D, EA), lambda i: (0, 0)),
        ],
        out_specs=[
            pl.BlockSpec((_NBLK, D), lambda i: (i, 0)),
            pl.BlockSpec((_NBLK, EA), lambda i: (i, 0)),
            pl.BlockSpec((_NBLK, EA), lambda i: (i, 0)),
        ],
        out_shape=[
            jax.ShapeDtypeStruct((N, D), jnp.float32),
            jax.ShapeDtypeStruct((N, EA), jnp.float32),
            jax.ShapeDtypeStruct((N, EA), jnp.float32),
        ],
    )(hw_p, sp_p, z, W2, We1, We2)


_EPT = E // _NS
_C = 80
_NCH = _EPT // _C
_DH = D // 2


def _edge_pass_body(ei_h, a2_h, wa3_h, tn2_h, ea_h,
                    hwp_h, spp_h,
                    a2_v, wa3_v, srcb, dstb, sdst, eab, gbuf, sbuf,
                    spb, hw_sh, sp_sh, gsem, rsem, psem, lsem, ssem):
    cid = lax.axis_index("c")
    sid = lax.axis_index("s")

    pltpu.sync_copy(a2_h, a2_v)
    pltpu.sync_copy(wa3_h, wa3_v)

    zz = jnp.zeros((16,), jnp.float32)
    for r in range(_C):
        for q in range(_DH // 16):
            sbuf[0, r, pl.ds(q * 16, 16)] = zz
        for q in range(_SP // 16):
            spb[0, r, pl.ds(q * 16, 16)] = zz
    for k in range(_NPS // _C):
        pltpu.sync_copy(sbuf.at[0], hw_sh.at[pl.ds(sid * _NPS + k * _C, _C), :])
        pltpu.sync_copy(spb.at[0], sp_sh.at[pl.ds(sid * _NPS + k * _C, _C), :])
    plsc.subcore_barrier()

    ebase = sid * _EPT

    def start_src(k, b):
        pltpu.make_async_copy(ei_h.at[0, pl.ds(ebase + k * _C, _C)],
                              srcb.at[b], ssem.at[b]).start()

    def drain_src(b):
        pltpu.make_async_copy(ei_h.at[0, pl.ds(0, _C)], srcb.at[b],
                              ssem.at[b]).wait()

    def start_lin(k, b):
        pltpu.make_async_copy(ei_h.at[1, pl.ds(ebase + k * _C, _C)],
                              dstb.at[b], lsem.at[b]).start()
        pltpu.make_async_copy(ea_h.at[pl.ds(ebase + k * _C, _C), :],
                              eab.at[b], lsem.at[b]).start()

    def drain_lin(b):
        pltpu.make_async_copy(ei_h.at[1, pl.ds(0, _C)], dstb.at[b],
                              lsem.at[b]).wait()
        pltpu.make_async_copy(ea_h.at[pl.ds(0, _C), :], eab.at[b],
                              lsem.at[b]).wait()

    def start_gather(bsrc, b):
        pltpu.make_async_copy(tn2_h.at[cid].at[srcb.at[bsrc]],
                              gbuf.at[b], gsem.at[b]).start()

    def drain_gather(b):
        pltpu.make_async_copy(tn2_h.at[cid].at[srcb.at[0]],
                              gbuf.at[b], gsem.at[b]).wait()

    def start_scat(b):
        pltpu.make_async_copy(sbuf.at[b], hw_sh.at[sdst.at[b]],
                              rsem.at[b]).start(add=True)

    def drain_scat(b):
        pltpu.make_async_copy(sbuf.at[b], hw_sh.at[sdst.at[0]],
                              rsem.at[b]).wait()

    def start_spscat(b):
        pltpu.make_async_copy(spb.at[b], sp_sh.at[sdst.at[b]],
                              psem.at[b]).start(add=True)

    def drain_spscat(b):
        pltpu.make_async_copy(spb.at[b], sp_sh.at[sdst.at[0]],
                              psem.at[b]).wait()

    def compute(b):
        w3 = wa3_v[...]
        w3s = [jnp.full((16,), w3[f], jnp.float32) for f in range(EA)]
        for g in range(_C // 16):
            sv = srcb[b, pl.ds(g * 16, 16)]
            dv = dstb[b, pl.ds(g * 16, 16)]
            sdst[b, pl.ds(g * 16, 16)] = dv
            av = plsc.load_gather(a2_v, [sv * 2])
            bv = plsc.load_gather(a2_v, [dv * 2 + 1])
            eids = lax.iota(jnp.int32, 16) + (g * 16)
            ee = jnp.zeros((16,), jnp.float32)
            for f in range(EA):
                fidx = jnp.full((16,), f, jnp.int32)
                col = plsc.load_gather(eab.at[b], [eids, fidx])
                ee = ee + col * w3s[f]
            ev = av + bv + ee
            ev = jnp.where(ev >= 0.0, ev, 0.1 * ev)
            pv = jnp.exp(ev)
            for l in range(16):
                e = g * 16 + l
                pvb = jnp.full((16,), pv[l], jnp.float32)
                for q in range(_DH // 16):
                    sbuf[b, e, pl.ds(q * 16, 16)] = (
                        gbuf[b, e, pl.ds(q * 16, 16)] * pvb)
                spb[b, e, pl.ds(0, 16)] = eab[b, e, :] * pvb
                spb[b, e, pl.ds(16, 16)] = pvb

    def half(k, b):
        @pl.when(k >= 2)
        def _():
            drain_scat(b)

        @pl.when((k >= 2) & (lax.rem(k, 2) == cid))
        def _():
            drain_spscat(b)

        @pl.when(k + 1 < _NCH)
        def _():
            start_lin(k + 1, 1 - b)

        drain_gather(b)

        @pl.when(k + 1 < _NCH)
        def _():
            drain_src(1 - b)
            start_gather(1 - b, 1 - b)

        drain_lin(b)
        compute(b)
        start_scat(b)

        @pl.when(lax.rem(k, 2) == cid)
        def _():
            start_spscat(b)

        @pl.when(k + 2 < _NCH)
        def _():
            start_src(k + 2, b)

    start_src(0, 0)
    start_src(1, 1)
    start_lin(0, 0)
    drain_src(0)
    start_gather(0, 0)

    def pair(t, carry):
        half(2 * t, 0)
        half(2 * t + 1, 1)
        return carry

    lax.fori_loop(0, _NCH // 2, pair, 0)

    drain_scat((_NCH - 1) % 2)
    drain_scat((_NCH - 2) % 2)

    @pl.when(cid == 0)
    def _():
        drain_spscat(0)

    @pl.when(cid == 1)
    def _():
        drain_spscat(1)

    plsc.subcore_barrier()

    pltpu.sync_copy(hw_sh.at[pl.ds(sid * _NPS, _NPS), :],
                    hwp_h.at[cid, pl.ds(sid * _NPS, _NPS), :])
    pltpu.sync_copy(sp_sh.at[pl.ds(sid * _NPS, _NPS), :],
                    spp_h.at[cid, pl.ds(sid * _NPS, _NPS), :])


def _edge_pass(ei, a2, wa3, t2, edge_attr):
    mesh = plsc.VectorSubcoreMesh(core_axis_name="c", subcore_axis_name="s")
    f = pl.kernel(
        _edge_pass_body,
        out_type=[
            jax.ShapeDtypeStruct((_NC, _NSH, _DH), jnp.float32),
            jax.ShapeDtypeStruct((_NC, _NSH, _SP), jnp.float32),
        ],
        mesh=mesh,
        scratch_types=[
            pltpu.VMEM((2 * N,), jnp.float32),
            pltpu.VMEM((EA,), jnp.float32),
            pltpu.VMEM((2, _C), jnp.int32),
            pltpu.VMEM((2, _C), jnp.int32),
            pltpu.VMEM((2, _C), jnp.int32),
            pltpu.VMEM((2, _C, EA), jnp.float32),
            pltpu.VMEM((2, _C, _DH), jnp.float32),
            pltpu.VMEM((2, _C, _DH), jnp.float32),
            pltpu.VMEM((2, _C, _SP), jnp.float32),
            pltpu.VMEM_SHARED((_NSH, _DH), jnp.float32),
            pltpu.VMEM_SHARED((_NSH, _SP), jnp.float32),
            pltpu.SemaphoreType.DMA((2,)),
            pltpu.SemaphoreType.DMA((2,)),
            pltpu.SemaphoreType.DMA((2,)),
            pltpu.SemaphoreType.DMA((2,)),
            pltpu.SemaphoreType.DMA((2,)),
        ],
        compiler_params=pltpu.CompilerParams(needs_layout_passes=False,
                                             use_tc_tiling_on_sc=False),
    )
    return f(ei, a2, wa3, t2, edge_attr)


_CW = 400
_NSUBW = _CW // 80
_NCHW = _EPT // _CW


def _wnew_body(ei_h, us_h, ud_h, wet_h, wnew_h,
               srcb, dstb, usr, udr, wetb, outb,
               gsem, lsem, ssem, wsem):
    sid = lax.axis_index("s")
    cid = lax.axis_index("c")
    ebase = (sid * _NC + cid) * (_EPT // _NC)
    nch = _NCHW // _NC

    def start_src(k, b):
        pltpu.make_async_copy(ei_h.at[0, pl.ds(ebase + k * _CW, _CW)],
                              srcb.at[b], ssem.at[b]).start()
        pltpu.make_async_copy(ei_h.at[1, pl.ds(ebase + k * _CW, _CW)],
                              dstb.at[b], ssem.at[b]).start()

    def drain_src(b):
        pltpu.make_async_copy(ei_h.at[0, pl.ds(0, _CW)], srcb.at[b],
                              ssem.at[b]).wait()
        pltpu.make_async_copy(ei_h.at[1, pl.ds(0, _CW)], dstb.at[b],
                              ssem.at[b]).wait()

    def start_lin(k, b):
        pltpu.make_async_copy(wet_h.at[pl.ds(ebase + k * _CW, _CW), :],
                              wetb.at[b], lsem.at[b]).start()

    def drain_lin(b):
        pltpu.make_async_copy(wet_h.at[pl.ds(0, _CW), :], wetb.at[b],
                              lsem.at[b]).wait()

    def start_gather(b):
        for j in range(_NSUBW):
            pltpu.make_async_copy(
                us_h.at[srcb.at[b, pl.ds(j * 80, 80)]],
                usr.at[b, pl.ds(j * 80, 80), :], gsem.at[b]).start()
            pltpu.make_async_copy(
                ud_h.at[dstb.at[b, pl.ds(j * 80, 80)]],
                udr.at[b, pl.ds(j * 80, 80), :], gsem.at[b]).start()

    def drain_gather(b):
        for j in range(_NSUBW):
            pltpu.make_async_copy(
                us_h.at[srcb.at[0, pl.ds(0, 80)]],
                usr.at[b, pl.ds(j * 80, 80), :], gsem.at[b]).wait()
            pltpu.make_async_copy(
                ud_h.at[dstb.at[0, pl.ds(0, 80)]],
                udr.at[b, pl.ds(j * 80, 80), :], gsem.at[b]).wait()

    def start_store(k, b):
        pltpu.make_async_copy(outb.at[b],
                              wnew_h.at[pl.ds(ebase + k * _CW, _CW), :],
                              wsem.at[b]).start()

    def drain_store(b):
        pltpu.make_async_copy(outb.at[b],
                              wnew_h.at[pl.ds(0, _CW), :],
                              wsem.at[b]).wait()

    def compute(b):
        for e in range(_CW):
            outb[b, e, :] = usr[b, e, :] + udr[b, e, :] + wetb[b, e, :]

    def half(k, b):
        @pl.when(k >= 2)
        def _():
            drain_store(b)

        @pl.when(k + 1 < nch)
        def _():
            start_lin(k + 1, 1 - b)

        drain_gather(b)

        @pl.when(k + 1 < nch)
        def _():
            drain_src(1 - b)
            start_gather(1 - b)

        drain_lin(b)
        compute(b)
        start_store(k, b)

        @pl.when(k + 2 < nch)
        def _():
            start_src(k + 2, b)

    start_src(0, 0)
    start_src(1, 1)
    start_lin(0, 0)
    drain_src(0)
    start_gather(0)

    def pair(t, carry):
        half(2 * t, 0)
        half(2 * t + 1, 1)
        return carry

    lax.fori_loop(0, nch // 2, pair, 0)
    if nch % 2:
        half(nch - 1, 0)

    drain_store((nch - 1) % 2)
    drain_store((nch - 2) % 2)


def _wnew(ei, u_src, u_dst, w_et):
    mesh = plsc.VectorSubcoreMesh(core_axis_name="c", subcore_axis_name="s")
    f = pl.kernel(
        _wnew_body,
        out_type=jax.ShapeDtypeStruct((E, EA), jnp.float32),
        mesh=mesh,
        scratch_types=[
            pltpu.VMEM((2, _CW), jnp.int32),
            pltpu.VMEM((2, _CW), jnp.int32),
            pltpu.VMEM((2, _CW, EA), jnp.float32),
            pltpu.VMEM((2, _CW, EA), jnp.float32),
            pltpu.VMEM((2, _CW, EA), jnp.float32),
            pltpu.VMEM((2, _CW, EA), jnp.float32),
            pltpu.SemaphoreType.DMA((2,)),
            pltpu.SemaphoreType.DMA((2,)),
            pltpu.SemaphoreType.DMA((2,)),
            pltpu.SemaphoreType.DMA((2,)),
        ],
        compiler_params=pltpu.CompilerParams(needs_layout_passes=False,
                                             use_tc_tiling_on_sc=False),
    )
    return f(ei, u_src, u_dst, w_et)


def kernel(x, edge_index, edge_attr, W_node, W_attn, W_to_node, W_edge):
    ei = edge_index.astype(jnp.int32)
    Wa = jnp.concatenate([W_attn[:D], W_attn[D:2 * D]], axis=1)
    z, t2, a2 = _node_mms(x, W_node, W_to_node[:D], Wa)
    w_et = _edge_mms(edge_attr, W_edge[2 * D:])

    hw_p, sp_p = _edge_pass(ei, a2.reshape(2 * N), W_attn[2 * D:, 0], t2,
                            edge_attr)

    h_new, u_src, u_dst = _combine(hw_p, sp_p, z,
                                   W_to_node[D:], W_edge[:D], W_edge[D:2 * D])

    w_new = _wnew(ei, u_src, u_dst, w_et)
    return h_new, w_new
